# Initial kernel scaffold; baseline (speedup 1.0000x reference)
import jax
import jax.numpy as jnp
from jax.experimental import pallas as pl


def kernel(x, W):
    raise NotImplementedError("write your pallas kernel here")


# speedup vs baseline: 3.6019x; 3.6019x over previous
import jax
import jax.numpy as jnp
from jax.experimental import pallas as pl
from jax.experimental.pallas import tpu as pltpu

_TOKENS = 16384
_DIM = 4096
_E = 64
_K = 8
_BR = 256


def _gate_kernel(x_ref, w_ref, fw_ref, idx_ref):
    logits = jax.lax.dot_general(
        x_ref[...], w_ref[...],
        dimension_numbers=(((1,), (1,)), ((), ())),
        preferred_element_type=jnp.float32,
    )

    lane = jax.lax.broadcasted_iota(jnp.int32, logits.shape, 1)
    neg_inf = jnp.float32(-jnp.inf)

    cur = logits
    sel = jnp.zeros(logits.shape, jnp.bool_)
    idx_cols = []
    mx = None
    denom = None
    for t in range(_K):
        m = jnp.max(cur, axis=1, keepdims=True)
        is_max = cur == m
        idx = jnp.min(jnp.where(is_max, lane, _E), axis=1, keepdims=True)
        chosen = lane == idx
        sel = sel | chosen
        cur = jnp.where(chosen, neg_inf, cur)
        idx_cols.append(idx)
        if t == 0:
            mx = m
            denom = jnp.ones(m.shape, jnp.float32)
        else:
            denom = denom + jnp.exp(m - mx)

    top_idx = jnp.concatenate(idx_cols, axis=1)
    inv = 1.0 / denom
    fw = jnp.where(sel, jnp.exp(logits - mx) * inv, 0.0)
    fw_ref[...] = fw
    idx_ref[...] = top_idx


@jax.jit
def kernel(x, W):
    grid = (_TOKENS // _BR,)
    fw, idx = pl.pallas_call(
        _gate_kernel,
        grid=grid,
        in_specs=[
            pl.BlockSpec((_BR, _DIM), lambda i: (i, 0)),
            pl.BlockSpec((_E, _DIM), lambda i: (0, 0)),
        ],
        out_specs=[
            pl.BlockSpec((_BR, _E), lambda i: (i, 0)),
            pl.BlockSpec((_BR, _K), lambda i: (i, 0)),
        ],
        out_shape=[
            jax.ShapeDtypeStruct((_TOKENS, _E), jnp.float32),
            jax.ShapeDtypeStruct((_TOKENS, _K), jnp.int32),
        ],
    )(x, W)
    return fw, idx


# speedup vs baseline: 5.0775x; 1.4097x over previous
import jax
import jax.numpy as jnp
from jax.experimental import pallas as pl
from jax.experimental.pallas import tpu as pltpu

_TOKENS = 16384
_DIM = 4096
_E = 64
_K = 8
_BR = 256


def _gate_kernel(x_ref, w_ref, fw_ref, idx_ref):
    logits_t = jax.lax.dot_general(
        w_ref[...], x_ref[...],
        dimension_numbers=(((1,), (1,)), ((), ())),
        preferred_element_type=jnp.float32,
    )

    erow = jax.lax.broadcasted_iota(jnp.int32, logits_t.shape, 0)
    neg_inf = jnp.float32(-jnp.inf)

    cur = logits_t
    sel = jnp.zeros(logits_t.shape, jnp.bool_)
    idx_rows = []
    mx = None
    denom = None
    for t in range(_K):
        m = jnp.max(cur, axis=0, keepdims=True)
        is_max = cur == m
        idx = jnp.min(jnp.where(is_max, erow, _E), axis=0, keepdims=True)
        chosen = erow == idx
        sel = sel | chosen
        cur = jnp.where(chosen, neg_inf, cur)
        idx_rows.append(idx)
        if t == 0:
            mx = m
            denom = jnp.ones(m.shape, jnp.float32)
        else:
            denom = denom + jnp.exp(m - mx)

    inv = 1.0 / denom
    fw_t = jnp.where(sel, jnp.exp(logits_t - mx) * inv, 0.0)
    idx_t = jnp.concatenate(idx_rows, axis=0)
    fw_ref[...] = fw_t.T
    idx_ref[...] = idx_t.T


@jax.jit
def kernel(x, W):
    grid = (_TOKENS // _BR,)
    fw, idx = pl.pallas_call(
        _gate_kernel,
        grid=grid,
        in_specs=[
            pl.BlockSpec((_BR, _DIM), lambda i: (i, 0)),
            pl.BlockSpec((_E, _DIM), lambda i: (0, 0)),
        ],
        out_specs=[
            pl.BlockSpec((_BR, _E), lambda i: (i, 0)),
            pl.BlockSpec((_BR, _K), lambda i: (i, 0)),
        ],
        out_shape=[
            jax.ShapeDtypeStruct((_TOKENS, _E), jnp.float32),
            jax.ShapeDtypeStruct((_TOKENS, _K), jnp.int32),
        ],
    )(x, W)
    return fw, idx


# speedup vs baseline: 6.0471x; 1.1910x over previous
import jax
import jax.numpy as jnp
from jax.experimental import pallas as pl
from jax.experimental.pallas import tpu as pltpu

_TOKENS = 16384
_DIM = 4096
_E = 64
_K = 8
_BR = 512


def _gate_kernel(x_ref, w_ref, fw_ref, idx_ref):
    logits_t = jax.lax.dot_general(
        w_ref[...], x_ref[...],
        dimension_numbers=(((1,), (1,)), ((), ())),
        preferred_element_type=jnp.float32,
    )

    erow = jax.lax.broadcasted_iota(jnp.int32, logits_t.shape, 0)
    neg_inf = jnp.float32(-jnp.inf)

    cur = logits_t
    sel = jnp.zeros(logits_t.shape, jnp.bool_)
    idx_rows = []
    mx = None
    denom = None
    for t in range(_K):
        m = jnp.max(cur, axis=0, keepdims=True)
        is_max = cur == m
        idx = jnp.min(jnp.where(is_max, erow, _E), axis=0, keepdims=True)
        chosen = erow == idx
        sel = sel | chosen
        cur = jnp.where(chosen, neg_inf, cur)
        idx_rows.append(idx)
        if t == 0:
            mx = m
            denom = jnp.ones(m.shape, jnp.float32)
        else:
            denom = denom + jnp.exp(m - mx)

    inv = 1.0 / denom
    fw_t = jnp.where(sel, jnp.exp(logits_t - mx) * inv, 0.0)
    idx_t = jnp.concatenate(idx_rows, axis=0)
    fw_ref[...] = fw_t.T
    idx_ref[...] = idx_t.T


@jax.jit
def kernel(x, W):
    grid = (_TOKENS // _BR,)
    fw, idx = pl.pallas_call(
        _gate_kernel,
        grid=grid,
        in_specs=[
            pl.BlockSpec((_BR, _DIM), lambda i: (i, 0)),
            pl.BlockSpec((_E, _DIM), lambda i: (0, 0)),
        ],
        out_specs=[
            pl.BlockSpec((_BR, _E), lambda i: (i, 0)),
            pl.BlockSpec((_BR, _K), lambda i: (i, 0)),
        ],
        out_shape=[
            jax.ShapeDtypeStruct((_TOKENS, _E), jnp.float32),
            jax.ShapeDtypeStruct((_TOKENS, _K), jnp.int32),
        ],
    )(x, W)
    return fw, idx


# speedup vs baseline: 6.4960x; 1.0742x over previous
import jax
import jax.numpy as jnp
from jax.experimental import pallas as pl
from jax.experimental.pallas import tpu as pltpu

_TOKENS = 16384
_DIM = 4096
_E = 64
_K = 8
_BR = 1024


def _gate_kernel(x_ref, w_ref, fw_ref, idx_ref):
    logits_t = jax.lax.dot_general(
        w_ref[...], x_ref[...],
        dimension_numbers=(((1,), (1,)), ((), ())),
        preferred_element_type=jnp.float32,
    )

    erow = jax.lax.broadcasted_iota(jnp.int32, logits_t.shape, 0)
    neg_inf = jnp.float32(-jnp.inf)

    cur = logits_t
    sel = jnp.zeros(logits_t.shape, jnp.bool_)
    idx_rows = []
    mx = None
    denom = None
    for t in range(_K):
        m = jnp.max(cur, axis=0, keepdims=True)
        is_max = cur == m
        idx = jnp.min(jnp.where(is_max, erow, _E), axis=0, keepdims=True)
        chosen = erow == idx
        sel = sel | chosen
        cur = jnp.where(chosen, neg_inf, cur)
        idx_rows.append(idx)
        if t == 0:
            mx = m
            denom = jnp.ones(m.shape, jnp.float32)
        else:
            denom = denom + jnp.exp(m - mx)

    inv = 1.0 / denom
    fw_t = jnp.where(sel, jnp.exp(logits_t - mx) * inv, 0.0)
    idx_t = jnp.concatenate(idx_rows, axis=0)
    fw_ref[...] = fw_t.T
    idx_ref[...] = idx_t.T


@jax.jit
def kernel(x, W):
    grid = (_TOKENS // _BR,)
    fw, idx = pl.pallas_call(
        _gate_kernel,
        grid=grid,
        in_specs=[
            pl.BlockSpec((_BR, _DIM), lambda i: (i, 0)),
            pl.BlockSpec((_E, _DIM), lambda i: (0, 0)),
        ],
        out_specs=[
            pl.BlockSpec((_BR, _E), lambda i: (i, 0)),
            pl.BlockSpec((_BR, _K), lambda i: (i, 0)),
        ],
        out_shape=[
            jax.ShapeDtypeStruct((_TOKENS, _E), jnp.float32),
            jax.ShapeDtypeStruct((_TOKENS, _K), jnp.int32),
        ],
    )(x, W)
    return fw, idx


# speedup vs baseline: 6.5013x; 1.0008x over previous
import jax
import jax.numpy as jnp
from jax.experimental import pallas as pl
from jax.experimental.pallas import tpu as pltpu

_TOKENS = 16384
_DIM = 4096
_E = 64
_K = 8
_BR = 1024


def _gate_kernel(x1_ref, x2_ref, w_ref, fw_ref, idx_ref):
    half = _DIM // 2
    logits_t = jax.lax.dot_general(
        w_ref[:, :half], x1_ref[...],
        dimension_numbers=(((1,), (1,)), ((), ())),
        preferred_element_type=jnp.float32,
    ) + jax.lax.dot_general(
        w_ref[:, half:], x2_ref[...],
        dimension_numbers=(((1,), (1,)), ((), ())),
        preferred_element_type=jnp.float32,
    )

    erow = jax.lax.broadcasted_iota(jnp.int32, logits_t.shape, 0)
    neg_inf = jnp.float32(-jnp.inf)

    cur = logits_t
    sel = jnp.zeros(logits_t.shape, jnp.bool_)
    idx_rows = []
    mx = None
    denom = None
    for t in range(_K):
        m = jnp.max(cur, axis=0, keepdims=True)
        is_max = cur == m
        idx = jnp.min(jnp.where(is_max, erow, _E), axis=0, keepdims=True)
        chosen = erow == idx
        sel = sel | chosen
        cur = jnp.where(chosen, neg_inf, cur)
        idx_rows.append(idx)
        if t == 0:
            mx = m
            denom = jnp.ones(m.shape, jnp.float32)
        else:
            denom = denom + jnp.exp(m - mx)

    inv = 1.0 / denom
    fw_t = jnp.where(sel, jnp.exp(logits_t - mx) * inv, 0.0)
    idx_t = jnp.concatenate(idx_rows, axis=0)
    fw_ref[...] = fw_t.T
    idx_ref[...] = idx_t.T


@jax.jit
def kernel(x, W):
    grid = (_TOKENS // _BR,)
    fw, idx = pl.pallas_call(
        _gate_kernel,
        grid=grid,
        in_specs=[
            pl.BlockSpec((_BR, _DIM // 2), lambda i: (i, 0)),
            pl.BlockSpec((_BR, _DIM // 2), lambda i: (i, 1)),
            pl.BlockSpec((_E, _DIM), lambda i: (0, 0)),
        ],
        out_specs=[
            pl.BlockSpec((_BR, _E), lambda i: (i, 0)),
            pl.BlockSpec((_BR, _K), lambda i: (i, 0)),
        ],
        out_shape=[
            jax.ShapeDtypeStruct((_TOKENS, _E), jnp.float32),
            jax.ShapeDtypeStruct((_TOKENS, _K), jnp.int32),
        ],
    )(x, x, W)
    return fw, idx
